# baseline (speedup 1.0000x reference)
import functools

import jax
import jax.numpy as jnp
from jax import lax
from jax.experimental import pallas as pl
from jax.experimental.pallas import tpu as pltpu
from jax.experimental.pallas import tpu_sc as plsc

NUM_SC = 2
NUM_TILES = 16
LANES = 128
CHUNK = 128


def _mesh():
    return plsc.VectorSubcoreMesh(core_axis_name="c", subcore_axis_name="s")


def _spmm(src2d, dst2d, s_flat, np_, p_total, nch):
    pps = p_total // NUM_SC
    rpt = np_ // NUM_TILES

    @functools.partial(
        pl.kernel,
        mesh=_mesh(),
        out_type=jax.ShapeDtypeStruct((p_total * np_, LANES), jnp.float32),
        scratch_types=[
            pltpu.VMEM((nch, CHUNK), jnp.int32),
            pltpu.VMEM((nch, CHUNK), jnp.int32),
            pltpu.VMEM((nch, CHUNK), jnp.int32),
            pltpu.VMEM((CHUNK, LANES), jnp.float32),
            pltpu.VMEM_SHARED((np_, LANES), jnp.float32),
            pltpu.SemaphoreType.DMA,
        ],
    )
    def spmmk(src_hbm, dst_hbm, s_hbm, out_hbm, srcv, srcadj, dstv, buf, acc,
              sem):
        c = lax.axis_index("c")
        sid = lax.axis_index("s")
        pltpu.sync_copy(src_hbm.at[sid], srcv)
        pltpu.sync_copy(dst_hbm.at[sid], dstv)
        for kk in range(pps):
            pbase = (c * pps + kk) * np_

            def adj(t, u):
                for l in range(CHUNK // 16):
                    sl = pl.ds(l * 16, 16)
                    srcadj[t, sl] = srcv[t, sl] + pbase
                return u

            lax.fori_loop(0, nch, adj, 0)
            pltpu.sync_copy(s_hbm.at[pl.ds(pbase + sid * rpt, rpt)],
                            acc.at[pl.ds(sid * rpt, rpt)])
            plsc.subcore_barrier()

            def edge(j, u):
                pltpu.async_copy(s_hbm.at[srcadj.at[j]], buf, sem).wait()
                pltpu.sync_copy(buf, acc.at[dstv.at[j]], add=True)
                return u

            lax.fori_loop(0, nch, edge, 0)
            plsc.subcore_barrier()
            pltpu.sync_copy(acc.at[pl.ds(sid * rpt, rpt)],
                            out_hbm.at[pl.ds(pbase + sid * rpt, rpt)])

    return spmmk(src2d, dst2d, s_flat)


def _scale(deg1, x, np_, r):
    p0 = x.shape[1] // LANES

    def body(d_ref, x_ref, dinv_ref, s0_ref):
        dinv = lax.rsqrt(d_ref[...])
        d1 = dinv[:, :1]
        dinv_ref[...] = dinv
        s0 = x_ref[...] * d1
        for p in range(p0):
            s0_ref[p] = s0[:, p * LANES:(p + 1) * LANES]
        for p in range(p0, 4):
            s0_ref[p] = jnp.zeros((r, LANES), jnp.float32)

    return pl.pallas_call(
        body,
        grid=(np_ // r,),
        in_specs=[
            pl.BlockSpec((r, LANES), lambda i: (i, 0)),
            pl.BlockSpec((r, x.shape[1]), lambda i: (i, 0)),
        ],
        out_specs=[
            pl.BlockSpec((r, LANES), lambda i: (i, 0)),
            pl.BlockSpec((4, r, LANES), lambda i: (0, i, 0)),
        ],
        out_shape=[
            jax.ShapeDtypeStruct((np_, LANES), jnp.float32),
            jax.ShapeDtypeStruct((4, np_, LANES), jnp.float32),
        ],
    )(deg1, x)


def _stage(agg, dinv, w, b, np_, r, relu_scale, p_in):
    d_out = w.shape[1]
    p_out = d_out // LANES

    def body(agg_ref, dinv_ref, w_ref, b_ref, out_ref):
        d1 = dinv_ref[...][:, :1]
        g = jnp.concatenate([agg_ref[p] for p in range(p_in)], axis=1) * d1
        acc = jnp.dot(g, w_ref[...], preferred_element_type=jnp.float32)
        acc = acc + b_ref[...]
        if relu_scale:
            acc = jnp.maximum(acc, 0.0) * d1
        for p in range(p_out):
            out_ref[p] = acc[:, p * LANES:(p + 1) * LANES]

    return pl.pallas_call(
        body,
        grid=(np_ // r,),
        in_specs=[
            pl.BlockSpec((p_in, r, LANES), lambda i: (0, i, 0)),
            pl.BlockSpec((r, LANES), lambda i: (i, 0)),
            pl.BlockSpec(w.shape, lambda i: (0, 0)),
            pl.BlockSpec((1, d_out), lambda i: (0, 0)),
        ],
        out_specs=pl.BlockSpec((p_out, r, LANES), lambda i: (0, i, 0)),
        out_shape=jax.ShapeDtypeStruct((p_out, np_, LANES), jnp.float32),
    )(agg, dinv, w, b.reshape(1, d_out))


def kernel(x, edge_index, W0, b0, W1, b1, W2, b2, W_mu, b_mu, W_ls, b_ls):
    n, d_in = x.shape
    e = edge_index.shape[1]
    np_ = -(-n // 128) * 128
    r = np_ // 16
    per_tile = NUM_TILES * CHUNK
    nch = -(-e // per_tile)
    epad = nch * per_tile

    src = edge_index[0].astype(jnp.int32)
    dst = edge_index[1].astype(jnp.int32)
    src2d = jnp.concatenate(
        [src, jnp.zeros((epad - e,), jnp.int32)]).reshape(NUM_TILES, nch,
                                                          CHUNK)
    dst2d = jnp.concatenate(
        [dst, jnp.full((epad - e,), n, jnp.int32)]).reshape(NUM_TILES, nch,
                                                            CHUNK)
    zsrc2d = jnp.zeros((NUM_TILES, nch, CHUNK), jnp.int32)
    xp = jnp.pad(x, ((0, np_ - n), (0, 0)))

    ones_op = jnp.ones((4 * np_, LANES), jnp.float32)
    deg1 = _spmm(zsrc2d, dst2d, ones_op, np_, 4, nch)[:np_]
    dinv, s0 = _scale(deg1, xp, np_, r)

    agg0 = _spmm(src2d, dst2d, s0.reshape(-1, LANES), np_, 4,
                 nch).reshape(4, np_, LANES)
    s1 = _stage(agg0, dinv, W0, b0, np_, r, True, d_in // LANES)

    agg1 = _spmm(src2d, dst2d, s1.reshape(-1, LANES), np_, 4,
                 nch).reshape(4, np_, LANES)
    s2 = _stage(agg1, dinv, W1, b1, np_, r, True, 4)

    agg2 = _spmm(src2d, dst2d, s2.reshape(-1, LANES), np_, 4,
                 nch).reshape(4, np_, LANES)
    s3 = _stage(agg2, dinv, W2, b2, np_, r, True, 4)

    agg3 = _spmm(src2d, dst2d, s3.reshape(-1, LANES), np_, 4,
                 nch).reshape(4, np_, LANES)
    w_cat = jnp.concatenate([W_mu, W_ls], axis=1)
    b_cat = jnp.concatenate([b_mu, b_ls])
    heads = _stage(agg3, dinv, w_cat, b_cat, np_, r, False, 4)
    return heads[0, :n], heads[1, :n]


# speedup vs baseline: 5.3277x; 5.3277x over previous
import functools

import jax
import jax.numpy as jnp
from jax import lax
from jax.experimental import pallas as pl
from jax.experimental.pallas import tpu as pltpu
from jax.experimental.pallas import tpu_sc as plsc

NUM_SC = 2
NUM_TILES = 16
LANES = 128
CHUNK = 128


def _mesh():
    return plsc.VectorSubcoreMesh(core_axis_name="c", subcore_axis_name="s")


def _spmm(src2d, dst2d, s_flat, np_, p_total, nch):
    pps = p_total // NUM_SC
    rpt = np_ // NUM_TILES

    @functools.partial(
        pl.kernel,
        mesh=_mesh(),
        out_type=jax.ShapeDtypeStruct((p_total * np_, LANES), jnp.float32),
        scratch_types=[
            pltpu.VMEM((nch, CHUNK), jnp.int32),
            pltpu.VMEM((nch, CHUNK), jnp.int32),
            pltpu.VMEM((nch, CHUNK), jnp.int32),
            pltpu.VMEM((CHUNK, LANES), jnp.float32),
            pltpu.VMEM_SHARED((np_, LANES), jnp.float32),
            pltpu.SemaphoreType.DMA,
        ],
    )
    def spmmk(src_hbm, dst_hbm, s_hbm, out_hbm, srcv, srcadj, dstv, buf, acc,
              sem):
        c = lax.axis_index("c")
        sid = lax.axis_index("s")
        pltpu.sync_copy(src_hbm.at[sid], srcv)
        pltpu.sync_copy(dst_hbm.at[sid], dstv)
        for kk in range(pps):
            pbase = (c * pps + kk) * np_

            def adj(t, u):
                for l in range(CHUNK // 16):
                    sl = pl.ds(l * 16, 16)
                    srcadj[t, sl] = srcv[t, sl] + pbase
                return u

            lax.fori_loop(0, nch, adj, 0)
            pltpu.sync_copy(s_hbm.at[pl.ds(pbase + sid * rpt, rpt)],
                            acc.at[pl.ds(sid * rpt, rpt)])
            plsc.subcore_barrier()

            def edge(j, u):
                pltpu.async_copy(s_hbm.at[srcadj.at[j]], buf, sem).wait()
                pltpu.sync_copy(buf, acc.at[dstv.at[j]], add=True)
                return u

            lax.fori_loop(0, nch, edge, 0)
            plsc.subcore_barrier()
            pltpu.sync_copy(acc.at[pl.ds(sid * rpt, rpt)],
                            out_hbm.at[pl.ds(pbase + sid * rpt, rpt)])

    return spmmk(src2d, dst2d, s_flat)


def _scale(deg1, x, np_, r):
    p0 = x.shape[1] // LANES

    def body(d_ref, x_ref, dinv_ref, s0_ref):
        dinv = lax.rsqrt(d_ref[...])
        d1 = dinv[:, :1]
        dinv_ref[...] = dinv
        s0 = x_ref[...] * d1
        for p in range(p0):
            s0_ref[p] = s0[:, p * LANES:(p + 1) * LANES]
        for p in range(p0, 4):
            s0_ref[p] = jnp.zeros((r, LANES), jnp.float32)

    return pl.pallas_call(
        body,
        grid=(np_ // r,),
        in_specs=[
            pl.BlockSpec((r, LANES), lambda i: (i, 0)),
            pl.BlockSpec((r, x.shape[1]), lambda i: (i, 0)),
        ],
        out_specs=[
            pl.BlockSpec((r, LANES), lambda i: (i, 0)),
            pl.BlockSpec((4, r, LANES), lambda i: (0, i, 0)),
        ],
        out_shape=[
            jax.ShapeDtypeStruct((np_, LANES), jnp.float32),
            jax.ShapeDtypeStruct((4, np_, LANES), jnp.float32),
        ],
    )(deg1, x)


def _stage(agg, dinv, w, b, np_, r, relu_scale, p_in):
    d_out = w.shape[1]
    p_out = d_out // LANES

    def body(agg_ref, dinv_ref, w_ref, b_ref, out_ref):
        d1 = dinv_ref[...][:, :1]
        g = jnp.concatenate([agg_ref[p] for p in range(p_in)], axis=1) * d1
        acc = jnp.dot(g, w_ref[...], preferred_element_type=jnp.float32)
        acc = acc + b_ref[...]
        if relu_scale:
            acc = jnp.maximum(acc, 0.0) * d1
        for p in range(p_out):
            out_ref[p] = acc[:, p * LANES:(p + 1) * LANES]

    return pl.pallas_call(
        body,
        grid=(np_ // r,),
        in_specs=[
            pl.BlockSpec((p_in, r, LANES), lambda i: (0, i, 0)),
            pl.BlockSpec((r, LANES), lambda i: (i, 0)),
            pl.BlockSpec(w.shape, lambda i: (0, 0)),
            pl.BlockSpec((1, d_out), lambda i: (0, 0)),
        ],
        out_specs=pl.BlockSpec((p_out, r, LANES), lambda i: (0, i, 0)),
        out_shape=jax.ShapeDtypeStruct((p_out, np_, LANES), jnp.float32),
    )(agg, dinv, w, b.reshape(1, d_out))


def kernel(x, edge_index, W0, b0, W1, b1, W2, b2, W_mu, b_mu, W_ls, b_ls):
    n, d_in = x.shape
    e = edge_index.shape[1]
    np_ = -(-n // 128) * 128
    r = np_ // 16
    per_tile = NUM_TILES * CHUNK
    nch = -(-e // per_tile)
    epad = nch * per_tile

    src = edge_index[0].astype(jnp.int32)
    dst = edge_index[1].astype(jnp.int32)
    src2d = jnp.concatenate(
        [src, jnp.zeros((epad - e,), jnp.int32)]).reshape(NUM_TILES, nch,
                                                          CHUNK)
    dst2d = jnp.concatenate(
        [dst, jnp.full((epad - e,), n, jnp.int32)]).reshape(NUM_TILES, nch,
                                                            CHUNK)
    xp = jnp.pad(x, ((0, np_ - n), (0, 0)))

    ones_op = jnp.ones((4 * np_, LANES), jnp.float32)
    deg1 = _spmm(dst2d, dst2d, ones_op, np_, 4, nch)[:np_]
    dinv, s0 = _scale(deg1, xp, np_, r)

    agg0 = _spmm(src2d, dst2d, s0.reshape(-1, LANES), np_, 4,
                 nch).reshape(4, np_, LANES)
    s1 = _stage(agg0, dinv, W0, b0, np_, r, True, d_in // LANES)

    agg1 = _spmm(src2d, dst2d, s1.reshape(-1, LANES), np_, 4,
                 nch).reshape(4, np_, LANES)
    s2 = _stage(agg1, dinv, W1, b1, np_, r, True, 4)

    agg2 = _spmm(src2d, dst2d, s2.reshape(-1, LANES), np_, 4,
                 nch).reshape(4, np_, LANES)
    s3 = _stage(agg2, dinv, W2, b2, np_, r, True, 4)

    agg3 = _spmm(src2d, dst2d, s3.reshape(-1, LANES), np_, 4,
                 nch).reshape(4, np_, LANES)
    w_cat = jnp.concatenate([W_mu, W_ls], axis=1)
    b_cat = jnp.concatenate([b_mu, b_ls])
    heads = _stage(agg3, dinv, w_cat, b_cat, np_, r, False, 4)
    return heads[0, :n], heads[1, :n]


# speedup vs baseline: 11.7356x; 2.2027x over previous
import functools

import jax
import jax.numpy as jnp
from jax import lax
from jax.experimental import pallas as pl
from jax.experimental.pallas import tpu as pltpu
from jax.experimental.pallas import tpu_sc as plsc

NUM_SC = 2
NUM_TILES = 16
LANES = 128
CHUNK = 128


def _mesh():
    return plsc.VectorSubcoreMesh(core_axis_name="c", subcore_axis_name="s")


def _spmm(src2d, dst2d, s_flat, np_, p_total, nch):
    pps = p_total // NUM_SC
    rpt = np_ // NUM_TILES
    nbuf = 2
    nhalf = nch // 2
    ngrp = nhalf // nbuf

    @functools.partial(
        pl.kernel,
        mesh=_mesh(),
        out_type=jax.ShapeDtypeStruct((p_total * np_, LANES), jnp.float32),
        scratch_types=[
            pltpu.VMEM((nhalf, CHUNK), jnp.int32),
            pltpu.VMEM((nhalf, CHUNK), jnp.int32),
            pltpu.VMEM((nbuf, CHUNK, LANES), jnp.float32),
            pltpu.VMEM_SHARED((np_, LANES), jnp.float32),
        ] + [pltpu.SemaphoreType.DMA] * nbuf,
    )
    def spmmk(src_hbm, dst_hbm, s_hbm, out_hbm, srcadj, dstv, bufs, acc,
              *gsem):
        c = lax.axis_index("c")
        sid = lax.axis_index("s")
        for kk in range(pps):
            pbase = (c * pps + kk) * np_
            pltpu.sync_copy(s_hbm.at[pl.ds(pbase + sid * rpt, rpt)],
                            acc.at[pl.ds(sid * rpt, rpt)])
            plsc.subcore_barrier()
            for h in range(2):
                pltpu.sync_copy(
                    src_hbm.at[sid, pl.ds(h * nhalf, nhalf)], srcadj)
                pltpu.sync_copy(
                    dst_hbm.at[sid, pl.ds(h * nhalf, nhalf)], dstv)

                def adj(t, u):
                    for l in range(CHUNK // 16):
                        sl = pl.ds(l * 16, 16)
                        srcadj[t, sl] = srcadj[t, sl] + pbase
                    return u

                lax.fori_loop(0, nhalf, adj, 0)
                for b in range(nbuf):
                    pltpu.async_copy(s_hbm.at[srcadj.at[b]], bufs.at[b],
                                     gsem[b])

                def grp(g, u):
                    j0 = g * nbuf
                    for b in range(nbuf):
                        j = j0 + b
                        pltpu.make_async_copy(s_hbm.at[srcadj.at[j]],
                                              bufs.at[b], gsem[b]).wait()
                        pltpu.sync_copy(bufs.at[b], acc.at[dstv.at[j]],
                                        add=True)
                        jn = j + nbuf

                        @pl.when(jn < nhalf)
                        def _():
                            pltpu.async_copy(s_hbm.at[srcadj.at[jn]],
                                             bufs.at[b], gsem[b])

                    return u

                lax.fori_loop(0, ngrp, grp, 0)
            plsc.subcore_barrier()
            pltpu.sync_copy(acc.at[pl.ds(sid * rpt, rpt)],
                            out_hbm.at[pl.ds(pbase + sid * rpt, rpt)])

    return spmmk(src2d, dst2d, s_flat)


def _scale(deg1, x, np_, r):
    p0 = x.shape[1] // LANES

    def body(d_ref, x_ref, dinv_ref, s0_ref):
        dinv = lax.rsqrt(d_ref[...])
        d1 = dinv[:, :1]
        dinv_ref[...] = dinv
        s0 = x_ref[...] * d1
        for p in range(p0):
            s0_ref[p] = s0[:, p * LANES:(p + 1) * LANES]
        for p in range(p0, 4):
            s0_ref[p] = jnp.zeros((r, LANES), jnp.float32)

    return pl.pallas_call(
        body,
        grid=(np_ // r,),
        in_specs=[
            pl.BlockSpec((r, LANES), lambda i: (i, 0)),
            pl.BlockSpec((r, x.shape[1]), lambda i: (i, 0)),
        ],
        out_specs=[
            pl.BlockSpec((r, LANES), lambda i: (i, 0)),
            pl.BlockSpec((4, r, LANES), lambda i: (0, i, 0)),
        ],
        out_shape=[
            jax.ShapeDtypeStruct((np_, LANES), jnp.float32),
            jax.ShapeDtypeStruct((4, np_, LANES), jnp.float32),
        ],
    )(deg1, x)


def _stage(agg, dinv, w, b, np_, r, relu_scale, p_in):
    d_out = w.shape[1]
    p_out = d_out // LANES

    def body(agg_ref, dinv_ref, w_ref, b_ref, out_ref):
        d1 = dinv_ref[...][:, :1]
        g = jnp.concatenate([agg_ref[p] for p in range(p_in)], axis=1) * d1
        acc = jnp.dot(g, w_ref[...], preferred_element_type=jnp.float32)
        acc = acc + b_ref[...]
        if relu_scale:
            acc = jnp.maximum(acc, 0.0) * d1
        for p in range(p_out):
            out_ref[p] = acc[:, p * LANES:(p + 1) * LANES]

    return pl.pallas_call(
        body,
        grid=(np_ // r,),
        in_specs=[
            pl.BlockSpec((p_in, r, LANES), lambda i: (0, i, 0)),
            pl.BlockSpec((r, LANES), lambda i: (i, 0)),
            pl.BlockSpec(w.shape, lambda i: (0, 0)),
            pl.BlockSpec((1, d_out), lambda i: (0, 0)),
        ],
        out_specs=pl.BlockSpec((p_out, r, LANES), lambda i: (0, i, 0)),
        out_shape=jax.ShapeDtypeStruct((p_out, np_, LANES), jnp.float32),
    )(agg, dinv, w, b.reshape(1, d_out))


def kernel(x, edge_index, W0, b0, W1, b1, W2, b2, W_mu, b_mu, W_ls, b_ls):
    n, d_in = x.shape
    e = edge_index.shape[1]
    np_ = -(-n // 128) * 128
    r = np_ // 16
    per_tile = NUM_TILES * CHUNK
    nch = -(-(-(-e // per_tile)) // 4) * 4
    epad = nch * per_tile

    pad_ar = jnp.arange(epad - e, dtype=jnp.int32)
    src = edge_index[0].astype(jnp.int32)
    dst = edge_index[1].astype(jnp.int32)
    src2d = jnp.concatenate(
        [src, pad_ar % n]).reshape(NUM_TILES, nch, CHUNK)
    dst2d = jnp.concatenate(
        [dst, n + pad_ar % (np_ - n)]).reshape(NUM_TILES, nch, CHUNK)
    xp = jnp.pad(x, ((0, np_ - n), (0, 0)))

    ones_op = jnp.ones((4 * np_, LANES), jnp.float32)
    deg1 = _spmm(dst2d, dst2d, ones_op, np_, 4, nch)[:np_]
    dinv, s0 = _scale(deg1, xp, np_, r)

    agg0 = _spmm(src2d, dst2d, s0.reshape(-1, LANES), np_, 4,
                 nch).reshape(4, np_, LANES)
    s1 = _stage(agg0, dinv, W0, b0, np_, r, True, d_in // LANES)

    agg1 = _spmm(src2d, dst2d, s1.reshape(-1, LANES), np_, 4,
                 nch).reshape(4, np_, LANES)
    s2 = _stage(agg1, dinv, W1, b1, np_, r, True, 4)

    agg2 = _spmm(src2d, dst2d, s2.reshape(-1, LANES), np_, 4,
                 nch).reshape(4, np_, LANES)
    s3 = _stage(agg2, dinv, W2, b2, np_, r, True, 4)

    agg3 = _spmm(src2d, dst2d, s3.reshape(-1, LANES), np_, 4,
                 nch).reshape(4, np_, LANES)
    w_cat = jnp.concatenate([W_mu, W_ls], axis=1)
    b_cat = jnp.concatenate([b_mu, b_ls])
    heads = _stage(agg3, dinv, w_cat, b_cat, np_, r, False, 4)
    return heads[0, :n], heads[1, :n]


# speedup vs baseline: 14.1627x; 1.2068x over previous
import functools

import jax
import jax.numpy as jnp
from jax import lax
from jax.experimental import pallas as pl
from jax.experimental.pallas import tpu as pltpu
from jax.experimental.pallas import tpu_sc as plsc

NUM_SC = 2
NUM_TILES = 16
LANES = 128
CHUNK = 128


def _mesh():
    return plsc.VectorSubcoreMesh(core_axis_name="c", subcore_axis_name="s")


def _spmm(src2d, dst2d, s_flat, np_, p_total, nch):
    pps = p_total // NUM_SC
    rpt = np_ // NUM_TILES
    nbuf = 2
    nhalf = nch // 2
    ngrp = nhalf // nbuf

    @functools.partial(
        pl.kernel,
        mesh=_mesh(),
        out_type=jax.ShapeDtypeStruct((p_total * np_, LANES), jnp.float32),
        scratch_types=[
            pltpu.VMEM((nhalf, CHUNK), jnp.int32),
            pltpu.VMEM((nhalf, CHUNK), jnp.int32),
            pltpu.VMEM((1, CHUNK), jnp.int32),
            pltpu.VMEM((nbuf, CHUNK, LANES), jnp.float32),
            pltpu.VMEM_SHARED((np_, LANES), jnp.float32),
        ] + [pltpu.SemaphoreType.DMA] * nbuf,
    )
    def spmmk(src_hbm, dst_hbm, s_hbm, out_hbm, srcadj, dstv, cfgv, bufs,
              acc, *gsem):
        c = lax.axis_index("c")
        sid = lax.axis_index("s")
        pltpu.sync_copy(src_hbm.at[sid, pl.ds(nch, 1)], cfgv)
        pps_eff = cfgv[0, pl.ds(0, 16)][0]
        for kk in range(pps):

            @pl.when(kk < pps_eff)
            def _pass():
                pbase = (c * pps_eff + kk) * np_
                pltpu.sync_copy(s_hbm.at[pl.ds(pbase + sid * rpt, rpt)],
                                acc.at[pl.ds(sid * rpt, rpt)])
                plsc.subcore_barrier()
                for h in range(2):
                    pltpu.sync_copy(
                        src_hbm.at[sid, pl.ds(h * nhalf, nhalf)], srcadj)
                    pltpu.sync_copy(
                        dst_hbm.at[sid, pl.ds(h * nhalf, nhalf)], dstv)

                    def adj(t, u):
                        for l in range(CHUNK // 16):
                            sl = pl.ds(l * 16, 16)
                            srcadj[t, sl] = srcadj[t, sl] + pbase
                        return u

                    lax.fori_loop(0, nhalf, adj, 0)
                    for b in range(nbuf):
                        pltpu.async_copy(s_hbm.at[srcadj.at[b]], bufs.at[b],
                                         gsem[b])

                    def grp(g, u):
                        j0 = g * nbuf
                        for b in range(nbuf):
                            j = j0 + b
                            pltpu.make_async_copy(s_hbm.at[srcadj.at[j]],
                                                  bufs.at[b], gsem[b]).wait()
                            pltpu.sync_copy(bufs.at[b], acc.at[dstv.at[j]],
                                            add=True)
                            jn = j + nbuf

                            @pl.when(jn < nhalf)
                            def _():
                                pltpu.async_copy(s_hbm.at[srcadj.at[jn]],
                                                 bufs.at[b], gsem[b])

                        return u

                    lax.fori_loop(0, ngrp, grp, 0)
                plsc.subcore_barrier()
                pltpu.sync_copy(acc.at[pl.ds(sid * rpt, rpt)],
                                out_hbm.at[pl.ds(pbase + sid * rpt, rpt)])

    return spmmk(src2d, dst2d, s_flat)


def _scale(deg1, x, np_, r):
    p0 = x.shape[1] // LANES

    def body(d_ref, x_ref, dinv_ref, s0_ref):
        dinv = lax.rsqrt(d_ref[...])
        d1 = dinv[:, :1]
        dinv_ref[...] = dinv
        s0 = x_ref[...] * d1
        for p in range(p0):
            s0_ref[p] = s0[:, p * LANES:(p + 1) * LANES]
        for p in range(p0, 4):
            s0_ref[p] = jnp.zeros((r, LANES), jnp.float32)

    return pl.pallas_call(
        body,
        grid=(np_ // r,),
        in_specs=[
            pl.BlockSpec((r, LANES), lambda i: (i, 0)),
            pl.BlockSpec((r, x.shape[1]), lambda i: (i, 0)),
        ],
        out_specs=[
            pl.BlockSpec((r, LANES), lambda i: (i, 0)),
            pl.BlockSpec((4, r, LANES), lambda i: (0, i, 0)),
        ],
        out_shape=[
            jax.ShapeDtypeStruct((np_, LANES), jnp.float32),
            jax.ShapeDtypeStruct((4, np_, LANES), jnp.float32),
        ],
    )(deg1, x)


def _stage(agg, dinv, w, b, np_, r, relu_scale, p_in):
    d_out = w.shape[1]
    p_out = d_out // LANES

    def body(agg_ref, dinv_ref, w_ref, b_ref, out_ref):
        d1 = dinv_ref[...][:, :1]
        g = jnp.concatenate([agg_ref[p] for p in range(p_in)], axis=1) * d1
        acc = jnp.dot(g, w_ref[...], preferred_element_type=jnp.float32)
        acc = acc + b_ref[...]
        if relu_scale:
            acc = jnp.maximum(acc, 0.0) * d1
        for p in range(p_out):
            out_ref[p] = acc[:, p * LANES:(p + 1) * LANES]

    return pl.pallas_call(
        body,
        grid=(np_ // r,),
        in_specs=[
            pl.BlockSpec((p_in, r, LANES), lambda i: (0, i, 0)),
            pl.BlockSpec((r, LANES), lambda i: (i, 0)),
            pl.BlockSpec(w.shape, lambda i: (0, 0)),
            pl.BlockSpec((1, d_out), lambda i: (0, 0)),
        ],
        out_specs=pl.BlockSpec((p_out, r, LANES), lambda i: (0, i, 0)),
        out_shape=jax.ShapeDtypeStruct((p_out, np_, LANES), jnp.float32),
    )(agg, dinv, w, b.reshape(1, d_out))


def kernel(x, edge_index, W0, b0, W1, b1, W2, b2, W_mu, b_mu, W_ls, b_ls):
    n, d_in = x.shape
    e = edge_index.shape[1]
    np_ = -(-n // 128) * 128
    r = np_ // 16
    per_tile = NUM_TILES * CHUNK
    nch = -(-(-(-e // per_tile)) // 4) * 4
    epad = nch * per_tile

    pad_ar = jnp.arange(epad - e, dtype=jnp.int32)
    src = edge_index[0].astype(jnp.int32)
    dst = edge_index[1].astype(jnp.int32)
    dst2d = jnp.concatenate(
        [dst, n + pad_ar % (np_ - n)]).reshape(NUM_TILES, nch, CHUNK)

    def with_cfg(slab2d, pps_eff):
        cfg = jnp.full((NUM_TILES, 1, CHUNK), pps_eff, jnp.int32)
        return jnp.concatenate([slab2d, cfg], axis=1)

    src2d_1 = with_cfg(jnp.concatenate(
        [src, pad_ar % n]).reshape(NUM_TILES, nch, CHUNK), 1)
    src2d_2 = src2d_1.at[:, nch].set(2)
    dstsrc_1 = with_cfg(dst2d, 1)
    xp = jnp.pad(x, ((0, np_ - n), (0, 0)))

    ones_op = jnp.ones((4 * np_, LANES), jnp.float32)
    deg1 = _spmm(dstsrc_1, dst2d, ones_op, np_, 4, nch)[:np_]
    dinv, s0 = _scale(deg1, xp, np_, r)

    agg0 = _spmm(src2d_1, dst2d, s0.reshape(-1, LANES), np_, 4,
                 nch).reshape(4, np_, LANES)
    s1 = _stage(agg0, dinv, W0, b0, np_, r, True, d_in // LANES)

    agg1 = _spmm(src2d_2, dst2d, s1.reshape(-1, LANES), np_, 4,
                 nch).reshape(4, np_, LANES)
    s2 = _stage(agg1, dinv, W1, b1, np_, r, True, 4)

    agg2 = _spmm(src2d_2, dst2d, s2.reshape(-1, LANES), np_, 4,
                 nch).reshape(4, np_, LANES)
    s3 = _stage(agg2, dinv, W2, b2, np_, r, True, 4)

    agg3 = _spmm(src2d_2, dst2d, s3.reshape(-1, LANES), np_, 4,
                 nch).reshape(4, np_, LANES)
    w_cat = jnp.concatenate([W_mu, W_ls], axis=1)
    b_cat = jnp.concatenate([b_mu, b_ls])
    heads = _stage(agg3, dinv, w_cat, b_cat, np_, r, False, 4)
    return heads[0, :n], heads[1, :n]
